# Initial kernel scaffold; baseline (speedup 1.0000x reference)
import jax
import jax.numpy as jnp
from jax.experimental import pallas as pl


def kernel(par_embedding, x, pos, batch, x_skip, pos_skip, batch_skip, W1, b1, W2, b2, Wp, bp):
    raise NotImplementedError("write your pallas kernel here")


# speedup vs baseline: 21.8631x; 21.8631x over previous
import jax
import jax.numpy as jnp
from jax import lax
from jax.experimental import pallas as pl
from jax.experimental.pallas import tpu as pltpu
from jax.experimental.pallas import tpu_sc as plsc

NC, NS, L = 2, 16, 16
NW = NC * NS
NQ = 16384
NX = 4096
QPW = NQ // NW
NG = QPW // L
CHUNK = 128
NCHUNK = QPW // CHUNK
D = 128
INF = float("inf")

_PERM_DN = lax.GatherDimensionNumbers(
    offset_dims=(), collapsed_slice_dims=(0,), start_index_map=(0,))


def _perm(v, idx):
    return lax.gather(v, idx[:, None], _PERM_DN, (1,),
                      mode=lax.GatherScatterMode.PROMISE_IN_BOUNDS)


def _ext(v, i):
    return lax.squeeze(lax.slice(v, (i,), (i + 1,)), (0,))


def _sc_knn_body(posx_h, posy_h, posz_h, qx_h, qy_h, qz_h, qb_h,
                 ss_h, se_h, glo_h, ghi_h, x_h, out_h,
                 posx, posy, posz, qx, qy, qz, qb, ssv_r, sev_r, glo, ghi,
                 wn1, wn2, wn3, i1, i2, i3, rows1, rows2, rows3, obuf, sem):
    wid = lax.axis_index("s") * NC + lax.axis_index("c")
    base = wid * QPW

    pltpu.sync_copy(posx_h, posx)
    pltpu.sync_copy(posy_h, posy)
    pltpu.sync_copy(posz_h, posz)
    pltpu.sync_copy(qx_h.at[pl.ds(base, QPW)], qx)
    pltpu.sync_copy(qy_h.at[pl.ds(base, QPW)], qy)
    pltpu.sync_copy(qz_h.at[pl.ds(base, QPW)], qz)
    pltpu.sync_copy(qb_h.at[pl.ds(base, QPW)], qb)
    pltpu.sync_copy(ss_h, ssv_r)
    pltpu.sync_copy(se_h, sev_r)
    pltpu.sync_copy(glo_h.at[pl.ds(wid * NG, NG)], glo.at[pl.ds(0, NG)])
    pltpu.sync_copy(ghi_h.at[pl.ds(wid * NG, NG)], ghi.at[pl.ds(0, NG)])

    ssv = ssv_r[pl.ds(0, L)]
    sev = sev_r[pl.ds(0, L)]

    @pl.loop(0, NG)
    def _group(g):
        gb = g * L
        qxg = qx[pl.ds(gb, L)]
        qyg = qy[pl.ds(gb, L)]
        qzg = qz[pl.ds(gb, L)]
        bg = qb[pl.ds(gb, L)]
        los = _perm(ssv, bg)
        his = _perm(sev, bg)
        lo = _ext(glo[pl.ds(g, L)], 0)
        hi = _ext(ghi[pl.ds(g, L)], 0)

        finf = jnp.full((L,), INF, jnp.float32)
        zi = jnp.zeros((L,), jnp.int32)

        @pl.loop(lo // L, (hi + L - 1) // L,
                 init_carry=(finf, finf, finf, zi, zi, zi))
        def _blk(jb, carry):
            d1, d2, d3, j1, j2, j3 = carry
            jb16 = jb * L
            px16 = posx[pl.ds(jb16, L)]
            py16 = posy[pl.ds(jb16, L)]
            pz16 = posz[pl.ds(jb16, L)]
            for t in range(L):
                tv = jnp.full((L,), t, jnp.int32)
                px = _perm(px16, tv)
                py = _perm(py16, tv)
                pz = _perm(pz16, tv)
                jv = jnp.full((L,), jb16 + t, jnp.int32)
                dx = qxg - px
                dy = qyg - py
                dz = qzg - pz
                d = dx * dx + dy * dy + dz * dz
                valid = (jv >= los) & (jv < his)
                dm = jnp.where(valid, d, INF)
                c1 = dm < d1
                c2 = dm < d2
                c3 = dm < d3
                d3n = jnp.where(c3, jnp.where(c2, d2, dm), d3)
                j3 = jnp.where(c3, jnp.where(c2, j2, jv), j3)
                d2n = jnp.where(c2, jnp.where(c1, d1, dm), d2)
                j2 = jnp.where(c2, jnp.where(c1, j1, jv), j2)
                d1 = jnp.where(c1, dm, d1)
                j1 = jnp.where(c1, jv, j1)
                d2, d3 = d2n, d3n
            return d1, d2, d3, j1, j2, j3

        d1, d2, d3, j1, j2, j3 = _blk
        w1 = 1.0 / jnp.maximum(d1, 1e-16)
        w2 = 1.0 / jnp.maximum(d2, 1e-16)
        w3 = 1.0 / jnp.maximum(d3, 1e-16)
        r = 1.0 / (w1 + w2 + w3)
        wn1[pl.ds(gb, L)] = w1 * r
        wn2[pl.ds(gb, L)] = w2 * r
        wn3[pl.ds(gb, L)] = w3 * r
        i1[pl.ds(gb, L)] = j1
        i2[pl.ds(gb, L)] = j2
        i3[pl.ds(gb, L)] = j3

    zsplat = jnp.zeros((L,), jnp.int32)
    for c in range(NCHUNK):
        cb = c * CHUNK
        cp1 = pltpu.async_copy(x_h.at[i1.at[pl.ds(cb, CHUNK)]], rows1, sem)
        cp2 = pltpu.async_copy(x_h.at[i2.at[pl.ds(cb, CHUNK)]], rows2, sem)
        cp3 = pltpu.async_copy(x_h.at[i3.at[pl.ds(cb, CHUNK)]], rows3, sem)
        cp1.wait()
        cp2.wait()
        cp3.wait()

        @pl.loop(0, CHUNK)
        def _combine(q):
            a1 = _perm(wn1[pl.ds(cb + q, L)], zsplat)
            a2 = _perm(wn2[pl.ds(cb + q, L)], zsplat)
            a3 = _perm(wn3[pl.ds(cb + q, L)], zsplat)
            for k in range(D // L):
                kk = pl.ds(k * L, L)
                obuf[q, kk] = (a1 * rows1[q, kk] + a2 * rows2[q, kk]
                               + a3 * rows3[q, kk])

        pltpu.sync_copy(obuf, out_h.at[pl.ds(base + cb, CHUNK)])


def _sc_knn(posx, posy, posz, qx, qy, qz, qb, ss, se, glo, ghi, x):
    mesh = plsc.VectorSubcoreMesh(core_axis_name="c", subcore_axis_name="s",
                                  num_cores=NC, num_subcores=NS)
    f = pl.kernel(
        _sc_knn_body,
        out_type=jax.ShapeDtypeStruct((NQ, D), jnp.float32),
        mesh=mesh,
        scratch_types=[
            pltpu.VMEM((NX,), jnp.float32),
            pltpu.VMEM((NX,), jnp.float32),
            pltpu.VMEM((NX,), jnp.float32),
            pltpu.VMEM((QPW,), jnp.float32),
            pltpu.VMEM((QPW,), jnp.float32),
            pltpu.VMEM((QPW,), jnp.float32),
            pltpu.VMEM((QPW,), jnp.int32),
            pltpu.VMEM((L,), jnp.int32),
            pltpu.VMEM((L,), jnp.int32),
            pltpu.VMEM((NG + L,), jnp.int32),
            pltpu.VMEM((NG + L,), jnp.int32),
            pltpu.VMEM((QPW + L,), jnp.float32),
            pltpu.VMEM((QPW + L,), jnp.float32),
            pltpu.VMEM((QPW + L,), jnp.float32),
            pltpu.VMEM((QPW,), jnp.int32),
            pltpu.VMEM((QPW,), jnp.int32),
            pltpu.VMEM((QPW,), jnp.int32),
            pltpu.VMEM((CHUNK, D), jnp.float32),
            pltpu.VMEM((CHUNK, D), jnp.float32),
            pltpu.VMEM((CHUNK, D), jnp.float32),
            pltpu.VMEM((CHUNK, D), jnp.float32),
            pltpu.SemaphoreType.DMA,
        ],
    )
    return f(posx, posy, posz, qx, qy, qz, qb, ss, se, glo, ghi, x)


def _tc_mlp_body(xi_ref, xs_ref, pe_ref, w1a_ref, w1b_ref, b1_ref,
                 w2_ref, b2_ref, wp_ref, bp_ref, out_ref):
    i = pl.program_id(0)
    pe_row = pe_ref[pl.ds(i, 1), :]
    pr = jnp.dot(pe_row, wp_ref[...],
                 preferred_element_type=jnp.float32) + bp_ref[...]
    pr = jnp.maximum(pr, 0.0)
    h = jnp.dot(xi_ref[...], w1a_ref[...], preferred_element_type=jnp.float32)
    h = h + jnp.dot(xs_ref[...], w1b_ref[...],
                    preferred_element_type=jnp.float32)
    h = jnp.maximum(h + b1_ref[...], 0.0)
    h = jnp.dot(h, w2_ref[...], preferred_element_type=jnp.float32)
    out_ref[...] = (h + b2_ref[...]) * pr


def _tc_mlp(xi, xs, pe, w1a, w1b, b1, w2, b2, wp, bp):
    nb = 8
    rb = NQ // nb
    return pl.pallas_call(
        _tc_mlp_body,
        grid=(nb,),
        in_specs=[
            pl.BlockSpec((rb, 128), lambda i: (i, 0)),
            pl.BlockSpec((rb, 64), lambda i: (i, 0)),
            pl.BlockSpec((8, 512), lambda i: (0, 0)),
            pl.BlockSpec((128, 128), lambda i: (0, 0)),
            pl.BlockSpec((64, 128), lambda i: (0, 0)),
            pl.BlockSpec((1, 128), lambda i: (0, 0)),
            pl.BlockSpec((128, 128), lambda i: (0, 0)),
            pl.BlockSpec((1, 128), lambda i: (0, 0)),
            pl.BlockSpec((512, 128), lambda i: (0, 0)),
            pl.BlockSpec((1, 128), lambda i: (0, 0)),
        ],
        out_specs=pl.BlockSpec((rb, 128), lambda i: (i, 0)),
        out_shape=jax.ShapeDtypeStruct((NQ, 128), jnp.float32),
    )(xi, xs, pe, w1a, w1b, b1, w2, b2, wp, bp)


def kernel(par_embedding, x, pos, batch, x_skip, pos_skip, batch_skip,
           W1, b1, W2, b2, Wp, bp):
    batch = batch.astype(jnp.int32)
    qb = batch_skip.astype(jnp.int32)
    posx = pos[:, 0] + 0.0
    posy = pos[:, 1] + 0.0
    posz = pos[:, 2] + 0.0
    qx = pos_skip[:, 0] + 0.0
    qy = pos_skip[:, 1] + 0.0
    qz = pos_skip[:, 2] + 0.0
    ar = jnp.arange(8, dtype=jnp.int32)
    ss = jnp.searchsorted(batch, ar, side="left").astype(jnp.int32)
    se = jnp.searchsorted(batch, ar, side="right").astype(jnp.int32)
    ss16 = jnp.pad(ss, (0, L - 8))
    se16 = jnp.pad(se, (0, L - 8))
    bs = qb.reshape(NQ // L, L)
    glo = jnp.pad(ss[bs[:, 0]], (0, L))
    ghi = jnp.pad(se[bs[:, L - 1]], (0, L))

    xi = _sc_knn(posx, posy, posz, qx, qy, qz, qb, ss16, se16, glo, ghi, x)

    pe = par_embedding.reshape(8, 512)
    w1a = W1[:128]
    w1b = W1[128:]
    out = _tc_mlp(xi, x_skip, pe, w1a, w1b, b1.reshape(1, 128),
                  W2, b2.reshape(1, 128), Wp, bp.reshape(1, 128))
    return out, pos_skip, batch_skip
